# Initial kernel scaffold; baseline (speedup 1.0000x reference)
import jax
import jax.numpy as jnp
from jax.experimental import pallas as pl


def kernel(inputs, weight):
    raise NotImplementedError("write your pallas kernel here")


# speedup vs baseline: 3.8045x; 3.8045x over previous
import jax
import jax.numpy as jnp
from jax.experimental import pallas as pl
from jax.experimental.pallas import tpu as pltpu

N_TOK = 16384
K = 1024
D = 2
T = 2048
G = N_TOK // T


def _vq_body(x_ref, w_ref, st_ref, perp_ref, loss_ref, hist_scr, loss_scr):
    i = pl.program_id(0)
    x = x_ref[...]
    w = w_ref[...]

    x2 = jnp.sum(x * x, axis=1, keepdims=True)
    w2 = jnp.sum(w * w, axis=1)
    m = jax.lax.dot_general(x, w, (((1,), (1,)), ((), ())),
                            preferred_element_type=jnp.float32)
    d = (x2 + w2[None, :]) - 2.0 * m

    mind = jnp.min(d, axis=1, keepdims=True)
    kio = jax.lax.broadcasted_iota(jnp.int32, (T, K), 1)
    idx = jnp.min(jnp.where(d == mind, kio, K), axis=1)

    onehot = (kio == idx[:, None]).astype(jnp.float32)
    q = jax.lax.dot_general(onehot, w, (((1,), (0,)), ((), ())),
                            preferred_element_type=jnp.float32)
    st_ref[...] = x + (q - x)

    hist_tile = jnp.sum(onehot, axis=0)[None, :]
    part = jnp.sum((q - x) ** 2)

    @pl.when(i == 0)
    def _init():
        hist_scr[...] = hist_tile
        loss_scr[0, 0] = part

    @pl.when(i > 0)
    def _acc():
        hist_scr[...] = hist_scr[...] + hist_tile
        loss_scr[0, 0] = loss_scr[0, 0] + part

    @pl.when(i == G - 1)
    def _finalize():
        avg = hist_scr[...] * (1.0 / N_TOK)
        ent = jnp.sum(avg * jnp.log(avg + 1e-10))
        perp_ref[...] = jnp.exp(-ent)[None, None]
        loss_ref[...] = (loss_scr[0, 0] * (1.0 / (N_TOK * D)))[None, None]


def kernel(inputs, weight):
    st, perp, loss = pl.pallas_call(
        _vq_body,
        grid=(G,),
        in_specs=[
            pl.BlockSpec((T, D), lambda i: (i, 0)),
            pl.BlockSpec((K, D), lambda i: (0, 0)),
        ],
        out_specs=[
            pl.BlockSpec((T, D), lambda i: (i, 0)),
            pl.BlockSpec((1, 1), lambda i: (0, 0)),
            pl.BlockSpec((1, 1), lambda i: (0, 0)),
        ],
        out_shape=[
            jax.ShapeDtypeStruct((N_TOK, D), jnp.float32),
            jax.ShapeDtypeStruct((1, 1), jnp.float32),
            jax.ShapeDtypeStruct((1, 1), jnp.float32),
        ],
        scratch_shapes=[
            pltpu.VMEM((1, K), jnp.float32),
            pltpu.SMEM((1, 1), jnp.float32),
        ],
    )(inputs, weight)
    return st, perp[0, 0], loss[0, 0]
